# baseline (speedup 1.0000x reference)
import functools

import jax
import jax.numpy as jnp
from jax import lax
from jax.experimental import pallas as pl
from jax.experimental.pallas import tpu as pltpu
from jax.experimental.pallas import tpu_sc as plsc

_UNITS = 32
_EPS = 1e-07
_NUM_SEG = 256
_N_TOK = 2048
_D_FEAT = 128
_NC = 2
_NS = 16
_NW = _NC * _NS
_F_SC = 32
_F_TC = _D_FEAT - _F_SC
_FPW = _F_SC // _NW
_FB_POST = 8
_FB_TC = 32


def _prep_kernel(xt_ref, idx_ref, lt_ref, ends_ref):
    eps = _EPS
    x = jnp.clip(xt_ref[...], eps, 1.0 - eps)
    lt_ref[0:_F_SC, :] = jnp.log(x)
    lt_ref[_F_SC : 2 * _F_SC, :] = jnp.log(1.0 - x)
    iota_s = lax.broadcasted_iota(jnp.int32, (_NUM_SEG, _N_TOK), 0)
    le = (idx_ref[:, :] <= iota_s).astype(jnp.int32)
    ends_ref[0, :] = jnp.sum(le, axis=1)


_GDN = lax.GatherDimensionNumbers(offset_dims=(), collapsed_slice_dims=(0,),
                                  start_index_map=(0,))


def _dyn_gather(v, idx_v):
    return lax.gather(v, idx_v[:, None], _GDN, (1,),
                      mode=lax.GatherScatterMode.PROMISE_IN_BOUNDS)


def _sc_accum(lt_hbm, ends_hbm, w_hbm, acc_hbm, lx_loc, l1_loc, ends_loc,
              w_loc, stage):
    wid = lax.axis_index("s") * _NC + lax.axis_index("c")
    f0 = wid * _FPW
    for f in range(_FPW):
        pltpu.sync_copy(lt_hbm.at[f0 + f],
                        lx_loc.at[pl.ds(f * _N_TOK, _N_TOK)])
        pltpu.sync_copy(lt_hbm.at[_F_SC + f0 + f],
                        l1_loc.at[pl.ds(f * _N_TOK, _N_TOK)])
    pltpu.sync_copy(ends_hbm, ends_loc)
    pltpu.sync_copy(w_hbm, w_loc)

    p = [[jnp.maximum(0.0, w_loc[2 * e + 1, pl.ds(16 * h, 16)])
          for h in range(2)] for e in range(4)]
    zero16 = jnp.zeros((16,), jnp.float32)
    lanes16 = lax.iota(jnp.int32, 16)
    zero16i = jnp.zeros((16,), jnp.int32)

    for f in range(_FPW):
        def seg_body(sidx, start, f=f):
            sa = (sidx // 16) * 16
            ev = ends_loc[pl.ds(sa, 16)]
            endf = jnp.sum(jnp.where(lanes16 == sidx - sa,
                                     ev.astype(jnp.float32), 0.0))
            end = endf.astype(jnp.int32)

            def tok_body(t, acc):
                ta = (t // 16) * 16
                lane_v = zero16i + (t - ta)
                lxb = _dyn_gather(lx_loc[pl.ds(f * _N_TOK + ta, 16)], lane_v)
                l1b = _dyn_gather(l1_loc[pl.ds(f * _N_TOK + ta, 16)], lane_v)
                new = []
                for e in range(4):
                    ls = lxb if e % 2 == 0 else l1b
                    for h in range(2):
                        new.append(acc[e * 2 + h] + jnp.exp(p[e][h] * ls))
                return tuple(new)

            acc = lax.fori_loop(start, end, tok_body, (zero16,) * 8)
            for e in range(4):
                for h in range(2):
                    stage[pl.ds(e * (_NUM_SEG * _UNITS) + sidx * _UNITS
                                + 16 * h, 16)] = acc[e * 2 + h]
            return end

        lax.fori_loop(0, _NUM_SEG, seg_body, 0)
        pltpu.sync_copy(stage, acc_hbm.at[f0 + f])


_sc_call = functools.partial(
    pl.kernel,
    out_type=jax.ShapeDtypeStruct((_F_SC, 4 * _NUM_SEG * _UNITS),
                                  jnp.float32),
    mesh=plsc.VectorSubcoreMesh(core_axis_name="c", subcore_axis_name="s",
                                num_cores=_NC, num_subcores=_NS),
    scratch_types=[
        pltpu.VMEM((_FPW * _N_TOK,), jnp.float32),
        pltpu.VMEM((_FPW * _N_TOK,), jnp.float32),
        pltpu.VMEM((_NUM_SEG,), jnp.int32),
        pltpu.VMEM((12, _UNITS), jnp.float32),
        pltpu.VMEM((4 * _NUM_SEG * _UNITS,), jnp.float32),
    ],
    compiler_params=pltpu.CompilerParams(needs_layout_passes=False),
)(_sc_accum)


def _post_kernel(_, acc_ref, w_ref, out_ref):
    eps = _EPS
    blk = _NUM_SEG * _UNITS

    def row128(r):
        return jnp.concatenate([w_ref[r : r + 1, :]] * 4, axis=1)

    q128 = [jnp.maximum(0.0, row128(2 * e)) for e in range(4)]
    ab128 = [row128(8 + e) for e in range(4)]
    for fl in range(_FB_POST):
        terms = []
        for e in range(4):
            a = jnp.reshape(acc_ref[fl : fl + 1, pl.ds(e * blk, blk)],
                            (64, 128))
            s = jnp.maximum(a, eps)
            terms.append(jnp.exp(q128[e] * jnp.log(s)) * ab128[e])
        num = terms[0] + terms[1]
        den = terms[2] + terms[3]
        mult = 2.0 * jnp.maximum(0.0, jnp.sign(den)) - 1.0
        den = jnp.where((den < eps) & (den > -eps), mult * eps, den)
        r64 = num / den
        for k in range(4):
            out_ref[slice(k, _NUM_SEG, 4), fl, :] = r64[:, 32 * k : 32 * k + 32]


def _tc_fused_kernel(xt_ref, idx_ref, w_ref, out_ref):
    eps = _EPS

    def col(row):
        return jnp.transpose(w_ref[row : row + 1, :])

    p_col = jnp.concatenate([jnp.maximum(0.0, col(r)) for r in (1, 3, 5, 7)],
                            axis=0)
    q_col = jnp.concatenate([jnp.maximum(0.0, col(r)) for r in (0, 2, 4, 6)],
                            axis=0)
    ab_col = jnp.concatenate([col(r) for r in (8, 9, 10, 11)], axis=0)

    idx_col = jnp.transpose(idx_ref[:, :])
    seg_iota = lax.broadcasted_iota(jnp.int32, (_N_TOK, _NUM_SEG), 1)
    oh_t = (idx_col == seg_iota).astype(jnp.float32)

    for f in range(_FB_TC):
        x = xt_ref[f : f + 1, :]
        x = jnp.clip(x, eps, 1.0 - eps)
        lx = jnp.log(x)
        l1 = jnp.log(1.0 - x)
        lx_b = jnp.broadcast_to(lx, (_UNITS, _N_TOK))
        l1_b = jnp.broadcast_to(l1, (_UNITS, _N_TOK))
        l_full = jnp.concatenate([lx_b, l1_b, lx_b, l1_b], axis=0)
        e_t = jnp.exp(p_col * l_full)
        acc = lax.dot(e_t, oh_t, preferred_element_type=jnp.float32)
        s = jnp.maximum(acc, eps)
        sq = jnp.exp(q_col * jnp.log(s))
        terms = sq * ab_col
        num = terms[0:32, :] + terms[32:64, :]
        den = terms[64:96, :] + terms[96:128, :]
        mult = 2.0 * jnp.maximum(0.0, jnp.sign(den)) - 1.0
        den = jnp.where((den < eps) & (den > -eps), mult * eps, den)
        res = num / den
        out_ref[:, f, :] = jnp.transpose(res)


@jax.jit
def kernel(inputs, index, w):
    xt = jnp.transpose(inputs)
    idx2d = index.reshape(1, _N_TOK)

    lt, ends2d = pl.pallas_call(
        _prep_kernel,
        grid=(1,),
        in_specs=[
            pl.BlockSpec((_F_SC, _N_TOK), lambda i: (0, 0)),
            pl.BlockSpec((1, _N_TOK), lambda i: (0, 0)),
        ],
        out_specs=[
            pl.BlockSpec((2 * _F_SC, _N_TOK), lambda i: (0, 0)),
            pl.BlockSpec((1, _NUM_SEG), lambda i: (0, 0)),
        ],
        out_shape=[
            jax.ShapeDtypeStruct((2 * _F_SC, _N_TOK), jnp.float32),
            jax.ShapeDtypeStruct((1, _NUM_SEG), jnp.int32),
        ],
    )(xt, idx2d)
    ends = ends2d.reshape(_NUM_SEG)
    acc = _sc_call(lt, ends, w)

    out_tc = pl.pallas_call(
        _tc_fused_kernel,
        grid=(_F_TC // _FB_TC,),
        in_specs=[
            pl.BlockSpec((_FB_TC, _N_TOK),
                         lambda i: (i + _F_SC // _FB_TC, 0)),
            pl.BlockSpec((1, _N_TOK), lambda i: (0, 0)),
            pl.BlockSpec((12, _UNITS), lambda i: (0, 0)),
        ],
        out_specs=pl.BlockSpec((_NUM_SEG, _FB_TC, _UNITS),
                               lambda i: (0, i + _F_SC // _FB_TC, 0)),
        out_shape=jax.ShapeDtypeStruct((_NUM_SEG, _D_FEAT, _UNITS),
                                       jnp.float32),
    )(xt, idx2d, w)

    out = pl.pallas_call(
        _post_kernel,
        grid=(_F_SC // _FB_POST,),
        in_specs=[
            pl.BlockSpec((_NUM_SEG, _FB_POST, _UNITS), lambda i: (0, i, 0)),
            pl.BlockSpec((_FB_POST, 4 * _NUM_SEG * _UNITS),
                         lambda i: (i, 0)),
            pl.BlockSpec((12, _UNITS), lambda i: (0, 0)),
        ],
        out_specs=pl.BlockSpec((_NUM_SEG, _FB_POST, _UNITS),
                               lambda i: (0, i, 0)),
        out_shape=jax.ShapeDtypeStruct((_NUM_SEG, _D_FEAT, _UNITS),
                                       jnp.float32),
        input_output_aliases={0: 0},
    )(out_tc, acc, w)
    return out


# speedup vs baseline: 1.0042x; 1.0042x over previous
import functools

import jax
import jax.numpy as jnp
from jax import lax
from jax.experimental import pallas as pl
from jax.experimental.pallas import tpu as pltpu
from jax.experimental.pallas import tpu_sc as plsc

_UNITS = 32
_EPS = 1e-07
_NUM_SEG = 256
_N_TOK = 2048
_D_FEAT = 128
_NC = 2
_NS = 16
_NW = _NC * _NS
_F_SC = 32
_F_TC = _D_FEAT - _F_SC
_FPW = _F_SC // _NW
_FB_POST = 8
_FB_TC = 16


def _prep_kernel(xt_ref, idx_ref, lt_ref, ends_ref):
    eps = _EPS
    x = jnp.clip(xt_ref[...], eps, 1.0 - eps)
    lt_ref[0:_F_SC, :] = jnp.log(x)
    lt_ref[_F_SC : 2 * _F_SC, :] = jnp.log(1.0 - x)
    iota_s = lax.broadcasted_iota(jnp.int32, (_NUM_SEG, _N_TOK), 0)
    le = (idx_ref[:, :] <= iota_s).astype(jnp.int32)
    ends_ref[0, :] = jnp.sum(le, axis=1)


_GDN = lax.GatherDimensionNumbers(offset_dims=(), collapsed_slice_dims=(0,),
                                  start_index_map=(0,))


def _dyn_gather(v, idx_v):
    return lax.gather(v, idx_v[:, None], _GDN, (1,),
                      mode=lax.GatherScatterMode.PROMISE_IN_BOUNDS)


def _sc_accum(lt_hbm, ends_hbm, w_hbm, acc_hbm, lx_loc, l1_loc, ends_loc,
              w_loc, stage):
    wid = lax.axis_index("s") * _NC + lax.axis_index("c")
    f0 = wid * _FPW
    for f in range(_FPW):
        pltpu.sync_copy(lt_hbm.at[f0 + f],
                        lx_loc.at[pl.ds(f * _N_TOK, _N_TOK)])
        pltpu.sync_copy(lt_hbm.at[_F_SC + f0 + f],
                        l1_loc.at[pl.ds(f * _N_TOK, _N_TOK)])
    pltpu.sync_copy(ends_hbm, ends_loc)
    pltpu.sync_copy(w_hbm, w_loc)

    p = [[jnp.maximum(0.0, w_loc[2 * e + 1, pl.ds(16 * h, 16)])
          for h in range(2)] for e in range(4)]
    zero16 = jnp.zeros((16,), jnp.float32)
    lanes16 = lax.iota(jnp.int32, 16)
    zero16i = jnp.zeros((16,), jnp.int32)

    for f in range(_FPW):
        def seg_body(sidx, start, f=f):
            sa = (sidx // 16) * 16
            ev = ends_loc[pl.ds(sa, 16)]
            endf = jnp.sum(jnp.where(lanes16 == sidx - sa,
                                     ev.astype(jnp.float32), 0.0))
            end = endf.astype(jnp.int32)

            def tok_body(t, acc):
                ta = (t // 16) * 16
                lane_v = zero16i + (t - ta)
                lxb = _dyn_gather(lx_loc[pl.ds(f * _N_TOK + ta, 16)], lane_v)
                l1b = _dyn_gather(l1_loc[pl.ds(f * _N_TOK + ta, 16)], lane_v)
                new = []
                for e in range(4):
                    ls = lxb if e % 2 == 0 else l1b
                    for h in range(2):
                        new.append(acc[e * 2 + h] + jnp.exp(p[e][h] * ls))
                return tuple(new)

            acc = lax.fori_loop(start, end, tok_body, (zero16,) * 8)
            for e in range(4):
                for h in range(2):
                    stage[pl.ds(e * (_NUM_SEG * _UNITS) + sidx * _UNITS
                                + 16 * h, 16)] = acc[e * 2 + h]
            return end

        lax.fori_loop(0, _NUM_SEG, seg_body, 0)
        pltpu.sync_copy(stage, acc_hbm.at[f0 + f])


_sc_call = functools.partial(
    pl.kernel,
    out_type=jax.ShapeDtypeStruct((_F_SC, 4 * _NUM_SEG * _UNITS),
                                  jnp.float32),
    mesh=plsc.VectorSubcoreMesh(core_axis_name="c", subcore_axis_name="s",
                                num_cores=_NC, num_subcores=_NS),
    scratch_types=[
        pltpu.VMEM((_FPW * _N_TOK,), jnp.float32),
        pltpu.VMEM((_FPW * _N_TOK,), jnp.float32),
        pltpu.VMEM((_NUM_SEG,), jnp.int32),
        pltpu.VMEM((12, _UNITS), jnp.float32),
        pltpu.VMEM((4 * _NUM_SEG * _UNITS,), jnp.float32),
    ],
    compiler_params=pltpu.CompilerParams(needs_layout_passes=False),
)(_sc_accum)


def _post_kernel(_, acc_ref, w_ref, out_ref):
    eps = _EPS
    blk = _NUM_SEG * _UNITS

    def row128(r):
        return jnp.concatenate([w_ref[r : r + 1, :]] * 4, axis=1)

    q128 = [jnp.maximum(0.0, row128(2 * e)) for e in range(4)]
    ab128 = [row128(8 + e) for e in range(4)]
    for fl in range(_FB_POST):
        terms = []
        for e in range(4):
            a = jnp.reshape(acc_ref[fl : fl + 1, pl.ds(e * blk, blk)],
                            (64, 128))
            s = jnp.maximum(a, eps)
            terms.append(jnp.exp(q128[e] * jnp.log(s)) * ab128[e])
        num = terms[0] + terms[1]
        den = terms[2] + terms[3]
        mult = 2.0 * jnp.maximum(0.0, jnp.sign(den)) - 1.0
        den = jnp.where((den < eps) & (den > -eps), mult * eps, den)
        r64 = num / den
        for k in range(4):
            out_ref[slice(k, _NUM_SEG, 4), fl, :] = r64[:, 32 * k : 32 * k + 32]


def _tc_fused_kernel(xt_ref, idx_ref, w_ref, out_ref):
    eps = _EPS

    def col(row):
        return jnp.transpose(w_ref[row : row + 1, :])

    p_col = jnp.concatenate([jnp.maximum(0.0, col(r)) for r in (1, 3, 5, 7)],
                            axis=0)
    q_col = jnp.concatenate([jnp.maximum(0.0, col(r)) for r in (0, 2, 4, 6)],
                            axis=0)
    ab_col = jnp.concatenate([col(r) for r in (8, 9, 10, 11)], axis=0)

    idx_col = jnp.transpose(idx_ref[:, :])
    seg_iota = lax.broadcasted_iota(jnp.int32, (_N_TOK, _NUM_SEG), 1)
    oh_t = (idx_col == seg_iota).astype(jnp.float32)

    for f in range(_FB_TC):
        x = xt_ref[f : f + 1, :]
        x = jnp.clip(x, eps, 1.0 - eps)
        lx = jnp.log(x)
        l1 = jnp.log(1.0 - x)
        lx_b = jnp.broadcast_to(lx, (_UNITS, _N_TOK))
        l1_b = jnp.broadcast_to(l1, (_UNITS, _N_TOK))
        l_full = jnp.concatenate([lx_b, l1_b, lx_b, l1_b], axis=0)
        e_t = jnp.exp(p_col * l_full)
        acc = lax.dot(e_t, oh_t, preferred_element_type=jnp.float32)
        s = jnp.maximum(acc, eps)
        sq = jnp.exp(q_col * jnp.log(s))
        terms = sq * ab_col
        num = terms[0:32, :] + terms[32:64, :]
        den = terms[64:96, :] + terms[96:128, :]
        mult = 2.0 * jnp.maximum(0.0, jnp.sign(den)) - 1.0
        den = jnp.where((den < eps) & (den > -eps), mult * eps, den)
        res = num / den
        out_ref[:, f, :] = jnp.transpose(res)


@jax.jit
def kernel(inputs, index, w):
    xt = jnp.transpose(inputs)
    idx2d = index.reshape(1, _N_TOK)

    lt, ends2d = pl.pallas_call(
        _prep_kernel,
        grid=(1,),
        in_specs=[
            pl.BlockSpec((_F_SC, _N_TOK), lambda i: (0, 0)),
            pl.BlockSpec((1, _N_TOK), lambda i: (0, 0)),
        ],
        out_specs=[
            pl.BlockSpec((2 * _F_SC, _N_TOK), lambda i: (0, 0)),
            pl.BlockSpec((1, _NUM_SEG), lambda i: (0, 0)),
        ],
        out_shape=[
            jax.ShapeDtypeStruct((2 * _F_SC, _N_TOK), jnp.float32),
            jax.ShapeDtypeStruct((1, _NUM_SEG), jnp.int32),
        ],
    )(xt, idx2d)
    ends = ends2d.reshape(_NUM_SEG)
    acc = _sc_call(lt, ends, w)

    out_tc = pl.pallas_call(
        _tc_fused_kernel,
        grid=(_F_TC // _FB_TC,),
        in_specs=[
            pl.BlockSpec((_FB_TC, _N_TOK),
                         lambda i: (i + _F_SC // _FB_TC, 0)),
            pl.BlockSpec((1, _N_TOK), lambda i: (0, 0)),
            pl.BlockSpec((12, _UNITS), lambda i: (0, 0)),
        ],
        out_specs=pl.BlockSpec((_NUM_SEG, _FB_TC, _UNITS),
                               lambda i: (0, i + _F_SC // _FB_TC, 0)),
        out_shape=jax.ShapeDtypeStruct((_NUM_SEG, _D_FEAT, _UNITS),
                                       jnp.float32),
    )(xt, idx2d, w)

    out = pl.pallas_call(
        _post_kernel,
        grid=(_F_SC // _FB_POST,),
        in_specs=[
            pl.BlockSpec((_NUM_SEG, _FB_POST, _UNITS), lambda i: (0, i, 0)),
            pl.BlockSpec((_FB_POST, 4 * _NUM_SEG * _UNITS),
                         lambda i: (i, 0)),
            pl.BlockSpec((12, _UNITS), lambda i: (0, 0)),
        ],
        out_specs=pl.BlockSpec((_NUM_SEG, _FB_POST, _UNITS),
                               lambda i: (0, i, 0)),
        out_shape=jax.ShapeDtypeStruct((_NUM_SEG, _D_FEAT, _UNITS),
                                       jnp.float32),
        input_output_aliases={0: 0},
    )(out_tc, acc, w)
    return out
